# baseline (speedup 1.0000x reference)
import dataclasses
import functools

import jax
import jax.numpy as jnp
from jax import lax
from jax.experimental import pallas as pl
from jax.experimental.pallas import tpu as pltpu
from jax.experimental.pallas import tpu_sc as plsc

N = 10000
D = 128
NC = 2
NS = 16
NW = NC * NS
CH = 120
NCHUNK = 84
E_PER_W = NCHUNK * CH
E_PAD = NW * E_PER_W
LANES = 16
ROWS_A = 624
ROWS_LAST = N - (NS - 1) * ROWS_A


def _matmul_body(x_ref, w_ref, o_ref):
    o_ref[...] = jnp.dot(x_ref[...], w_ref[...],
                         preferred_element_type=jnp.float32)


def _matmul(x, W):
    m_blk = 1000
    return pl.pallas_call(
        _matmul_body,
        grid=(N // m_blk,),
        in_specs=[
            pl.BlockSpec((m_blk, D), lambda i: (i, 0)),
            pl.BlockSpec((D, D), lambda i: (0, 0)),
        ],
        out_specs=pl.BlockSpec((m_blk, D), lambda i: (i, 0)),
        out_shape=jax.ShapeDtypeStruct((N, D), jnp.float32),
    )(x, W)


def _sc_body(xw_hbm, pk_hbm, zeros_hbm, out_hbm,
             pk_v, rows_v, acc_sh,
             sem_i0, sem_i1, sem_i2, sem_i3,
             sem_g0, sem_g1, sem_g2,
             sem_s0, sem_s1, sem_s2):
    cid = lax.axis_index("c")
    sid = lax.axis_index("s")
    wid = sid * NC + cid

    r0 = sid * ROWS_A

    @pl.when(sid < NS - 1)
    def _():
        pltpu.sync_copy(zeros_hbm.at[pl.ds(r0, ROWS_A)],
                        acc_sh.at[pl.ds(r0, ROWS_A)])

    @pl.when(sid == NS - 1)
    def _():
        pltpu.sync_copy(zeros_hbm.at[pl.ds((NS - 1) * ROWS_A, ROWS_LAST)],
                        acc_sh.at[pl.ds((NS - 1) * ROWS_A, ROWS_LAST)])

    plsc.subcore_barrier()

    c0 = 144
    c1 = 2 * NCHUNK - c0
    nch = jnp.where(cid == 0, c0, c1)
    base = jnp.where(cid == 0, sid * c0, NS * c0 + sid * c1)
    sem_i = (sem_i0, sem_i1, sem_i2, sem_i3)
    sem_g = (sem_g0, sem_g1, sem_g2)
    sem_s = (sem_s0, sem_s1, sem_s2)

    def wait_idx(p):
        pltpu.make_async_copy(pk_hbm.at[base], pk_v.at[p], sem_i[p]).wait()

    def wait_gather(r):
        pltpu.make_async_copy(xw_hbm.at[pk_v.at[r, 0]], rows_v.at[r],
                              sem_g[r]).wait()

    def wait_scatter(r):
        pltpu.make_async_copy(rows_v.at[r], acc_sh.at[pk_v.at[r, 1]],
                              sem_s[r]).wait()

    pltpu.async_copy(pk_hbm.at[base], pk_v.at[0], sem_i0)
    pltpu.async_copy(pk_hbm.at[base + 1], pk_v.at[1], sem_i1)
    pltpu.async_copy(xw_hbm.at[pl.ds(0, CH)], rows_v.at[1], sem_s1)
    pltpu.async_copy(xw_hbm.at[pl.ds(0, CH)], rows_v.at[2], sem_s2)
    wait_idx(0)
    pltpu.async_copy(xw_hbm.at[pk_v.at[0, 0]], rows_v.at[0], sem_g0)

    @pl.loop(0, nch, step=12)
    def _chunk(g):
        for u in range(12):
            gg = g + u
            r = u % 3
            r1 = (u + 1) % 3
            p = u % 4
            p1 = (u + 1) % 4
            p2 = (u + 2) % 4
            wait_idx(p1)
            wait_scatter(r1)
            pltpu.async_copy(xw_hbm.at[pk_v.at[p1, 0]], rows_v.at[r1],
                             sem_g[r1])
            pltpu.async_copy(pk_hbm.at[base + gg + 2], pk_v.at[p2],
                             sem_i[p2])
            wait_gather(r)

            iota16 = lax.iota(jnp.int32, LANES)

            @pl.loop(0, CH - CH % LANES, step=LANES)
            def _scale(k0):
                wvec = plsc.bitcast(
                    plsc.load_gather(pk_v.at[p, 2], [k0 + iota16]),
                    jnp.float32)
                for j in range(LANES):
                    wspl = lax.broadcast(wvec[j], (LANES,))
                    for c in range(D // LANES):
                        sl = pl.ds(c * LANES, LANES)
                        rows_v[r, k0 + j, sl] = rows_v[r, k0 + j, sl] * wspl

            if CH % LANES:
                t0 = CH - LANES
                wvec_t = plsc.bitcast(
                    plsc.load_gather(pk_v.at[p, 2], [t0 + iota16]),
                    jnp.float32)
                for j in range(LANES - CH % LANES, LANES):
                    wspl = lax.broadcast(wvec_t[j], (LANES,))
                    for c in range(D // LANES):
                        sl = pl.ds(c * LANES, LANES)
                        rows_v[r, t0 + j, sl] = rows_v[r, t0 + j, sl] * wspl

            pltpu.async_copy(rows_v.at[r], acc_sh.at[pk_v.at[p, 1]],
                             sem_s[r], add=True)

    wait_scatter((NCHUNK - 2) % 3)
    wait_scatter((NCHUNK - 1) % 3)
    wait_gather(NCHUNK % 3)
    wait_idx((NCHUNK + 1) % 4)

    plsc.subcore_barrier()

    @pl.when(sid < NS - 1)
    def _():
        pltpu.sync_copy(acc_sh.at[pl.ds(r0, ROWS_A)],
                        out_hbm.at[cid, pl.ds(r0, ROWS_A)])

    @pl.when(sid == NS - 1)
    def _():
        pltpu.sync_copy(acc_sh.at[pl.ds((NS - 1) * ROWS_A, ROWS_LAST)],
                        out_hbm.at[cid, pl.ds((NS - 1) * ROWS_A, ROWS_LAST)])


def _sc_spmm(xw, packed, zeros):
    mesh = plsc.VectorSubcoreMesh(core_axis_name="c", subcore_axis_name="s")
    cp = pltpu.CompilerParams()
    if "needs_layout_passes" in pltpu.CompilerParams.__dataclass_fields__:
        cp = dataclasses.replace(cp, needs_layout_passes=False)
    run = pl.kernel(
        _sc_body,
        mesh=mesh,
        compiler_params=cp,
        out_type=jax.ShapeDtypeStruct((NC, N, D), jnp.float32),
        scratch_types=[
            pltpu.VMEM((4, 3, CH), jnp.int32),
            pltpu.VMEM((3, CH, D), jnp.float32),
            pltpu.VMEM_SHARED((N, D), jnp.float32),
        ] + [pltpu.SemaphoreType.DMA] * 10,
    )
    return run(xw, packed, zeros)


def _combine_body(p_ref, b_ref, o_ref):
    o_ref[...] = p_ref[0] + p_ref[1] + b_ref[...]


def _combine(partials, b2d):
    m_blk = 1000
    return pl.pallas_call(
        _combine_body,
        grid=(N // m_blk,),
        in_specs=[
            pl.BlockSpec((NC, m_blk, D), lambda i: (0, i, 0)),
            pl.BlockSpec((1, D), lambda i: (0, 0)),
        ],
        out_specs=pl.BlockSpec((m_blk, D), lambda i: (i, 0)),
        out_shape=jax.ShapeDtypeStruct((N, D), jnp.float32),
    )(partials, b2d)


def kernel(x, edge_index, edge_weight, W, b):
    src = edge_index[0].astype(jnp.int32)
    dst = edge_index[1].astype(jnp.int32)
    wbits = lax.bitcast_convert_type(edge_weight.astype(jnp.float32),
                                     jnp.int32)

    pad = E_PAD - src.shape[0]
    zi = jnp.zeros((pad,), jnp.int32)
    src = jnp.concatenate([src, zi])
    dst = jnp.concatenate([dst, zi])
    wbits = jnp.concatenate([wbits, zi])

    packed = jnp.stack([src, dst, wbits], axis=0).reshape(3, -1, CH)
    packed = jnp.swapaxes(packed, 0, 1)
    packed = jnp.concatenate(
        [packed, jnp.zeros((2, 3, CH), jnp.int32)], axis=0)

    xw = _matmul(x, W)
    zeros = jnp.zeros((N, D), jnp.float32)
    partials = _sc_spmm(xw, packed, zeros)
    return _combine(partials, b.reshape(1, D))


# speedup vs baseline: 1.0290x; 1.0290x over previous
import dataclasses
import functools

import jax
import jax.numpy as jnp
from jax import lax
from jax.experimental import pallas as pl
from jax.experimental.pallas import tpu as pltpu
from jax.experimental.pallas import tpu_sc as plsc

N = 10000
D = 128
NC = 2
NS = 16
NW = NC * NS
CH = 120
NCHUNK = 84
E_PER_W = NCHUNK * CH
E_PAD = NW * E_PER_W
LANES = 16
ROWS_A = 624
ROWS_LAST = N - (NS - 1) * ROWS_A


def _matmul_body(x_ref, w_ref, o_ref):
    o_ref[...] = jnp.dot(x_ref[...], w_ref[...],
                         preferred_element_type=jnp.float32)


def _matmul(x, W):
    m_blk = 1000
    return pl.pallas_call(
        _matmul_body,
        grid=(N // m_blk,),
        in_specs=[
            pl.BlockSpec((m_blk, D), lambda i: (i, 0)),
            pl.BlockSpec((D, D), lambda i: (0, 0)),
        ],
        out_specs=pl.BlockSpec((m_blk, D), lambda i: (i, 0)),
        out_shape=jax.ShapeDtypeStruct((N, D), jnp.float32),
    )(x, W)


def _sc_body(xw_hbm, pk_hbm, zeros_hbm, out_hbm,
             pk_v, rows_v, acc_sh,
             sem_i0, sem_i1, sem_i2, sem_i3,
             sem_g0, sem_g1, sem_g2,
             sem_s0, sem_s1, sem_s2):
    cid = lax.axis_index("c")
    sid = lax.axis_index("s")
    wid = sid * NC + cid

    r0 = sid * ROWS_A

    @pl.when(sid < NS - 1)
    def _():
        pltpu.sync_copy(zeros_hbm.at[pl.ds(r0, ROWS_A)],
                        acc_sh.at[pl.ds(r0, ROWS_A)])

    @pl.when(sid == NS - 1)
    def _():
        pltpu.sync_copy(zeros_hbm.at[pl.ds((NS - 1) * ROWS_A, ROWS_LAST)],
                        acc_sh.at[pl.ds((NS - 1) * ROWS_A, ROWS_LAST)])

    plsc.subcore_barrier()

    c0 = 132
    c1 = 2 * NCHUNK - c0
    nch = jnp.where(cid == 0, c0, c1)
    base = jnp.where(cid == 0, sid * c0, NS * c0 + sid * c1)
    sem_i = (sem_i0, sem_i1, sem_i2, sem_i3)
    sem_g = (sem_g0, sem_g1, sem_g2)
    sem_s = (sem_s0, sem_s1, sem_s2)

    def wait_idx(p):
        pltpu.make_async_copy(pk_hbm.at[base], pk_v.at[p], sem_i[p]).wait()

    def wait_gather(r):
        pltpu.make_async_copy(xw_hbm.at[pk_v.at[r, 0]], rows_v.at[r],
                              sem_g[r]).wait()

    def wait_scatter(r):
        pltpu.make_async_copy(rows_v.at[r], acc_sh.at[pk_v.at[r, 1]],
                              sem_s[r]).wait()

    pltpu.async_copy(pk_hbm.at[base], pk_v.at[0], sem_i0)
    pltpu.async_copy(pk_hbm.at[base + 1], pk_v.at[1], sem_i1)
    pltpu.async_copy(xw_hbm.at[pl.ds(0, CH)], rows_v.at[1], sem_s1)
    pltpu.async_copy(xw_hbm.at[pl.ds(0, CH)], rows_v.at[2], sem_s2)
    wait_idx(0)
    pltpu.async_copy(xw_hbm.at[pk_v.at[0, 0]], rows_v.at[0], sem_g0)

    @pl.loop(0, nch, step=12)
    def _chunk(g):
        for u in range(12):
            gg = g + u
            r = u % 3
            r1 = (u + 1) % 3
            p = u % 4
            p1 = (u + 1) % 4
            p2 = (u + 2) % 4
            wait_idx(p1)
            wait_scatter(r1)
            pltpu.async_copy(xw_hbm.at[pk_v.at[p1, 0]], rows_v.at[r1],
                             sem_g[r1])
            pltpu.async_copy(pk_hbm.at[base + gg + 2], pk_v.at[p2],
                             sem_i[p2])
            wait_gather(r)

            iota16 = lax.iota(jnp.int32, LANES)

            @pl.loop(0, CH - CH % LANES, step=LANES)
            def _scale(k0):
                wvec = plsc.bitcast(
                    plsc.load_gather(pk_v.at[p, 2], [k0 + iota16]),
                    jnp.float32)
                for j in range(LANES):
                    wspl = lax.broadcast(wvec[j], (LANES,))
                    for c in range(D // LANES):
                        sl = pl.ds(c * LANES, LANES)
                        rows_v[r, k0 + j, sl] = rows_v[r, k0 + j, sl] * wspl

            if CH % LANES:
                t0 = CH - LANES
                wvec_t = plsc.bitcast(
                    plsc.load_gather(pk_v.at[p, 2], [t0 + iota16]),
                    jnp.float32)
                for j in range(LANES - CH % LANES, LANES):
                    wspl = lax.broadcast(wvec_t[j], (LANES,))
                    for c in range(D // LANES):
                        sl = pl.ds(c * LANES, LANES)
                        rows_v[r, t0 + j, sl] = rows_v[r, t0 + j, sl] * wspl

            pltpu.async_copy(rows_v.at[r], acc_sh.at[pk_v.at[p, 1]],
                             sem_s[r], add=True)

    wait_scatter((NCHUNK - 2) % 3)
    wait_scatter((NCHUNK - 1) % 3)
    wait_gather(NCHUNK % 3)
    wait_idx((NCHUNK + 1) % 4)

    plsc.subcore_barrier()

    @pl.when(sid < NS - 1)
    def _():
        pltpu.sync_copy(acc_sh.at[pl.ds(r0, ROWS_A)],
                        out_hbm.at[cid, pl.ds(r0, ROWS_A)])

    @pl.when(sid == NS - 1)
    def _():
        pltpu.sync_copy(acc_sh.at[pl.ds((NS - 1) * ROWS_A, ROWS_LAST)],
                        out_hbm.at[cid, pl.ds((NS - 1) * ROWS_A, ROWS_LAST)])


def _sc_spmm(xw, packed, zeros):
    mesh = plsc.VectorSubcoreMesh(core_axis_name="c", subcore_axis_name="s")
    cp = pltpu.CompilerParams()
    if "needs_layout_passes" in pltpu.CompilerParams.__dataclass_fields__:
        cp = dataclasses.replace(cp, needs_layout_passes=False)
    run = pl.kernel(
        _sc_body,
        mesh=mesh,
        compiler_params=cp,
        out_type=jax.ShapeDtypeStruct((NC, N, D), jnp.float32),
        scratch_types=[
            pltpu.VMEM((4, 3, CH), jnp.int32),
            pltpu.VMEM((3, CH, D), jnp.float32),
            pltpu.VMEM_SHARED((N, D), jnp.float32),
        ] + [pltpu.SemaphoreType.DMA] * 10,
    )
    return run(xw, packed, zeros)


def _combine_body(p_ref, b_ref, o_ref):
    o_ref[...] = p_ref[0] + p_ref[1] + b_ref[...]


def _combine(partials, b2d):
    m_blk = 1000
    return pl.pallas_call(
        _combine_body,
        grid=(N // m_blk,),
        in_specs=[
            pl.BlockSpec((NC, m_blk, D), lambda i: (0, i, 0)),
            pl.BlockSpec((1, D), lambda i: (0, 0)),
        ],
        out_specs=pl.BlockSpec((m_blk, D), lambda i: (i, 0)),
        out_shape=jax.ShapeDtypeStruct((N, D), jnp.float32),
    )(partials, b2d)


def kernel(x, edge_index, edge_weight, W, b):
    src = edge_index[0].astype(jnp.int32)
    dst = edge_index[1].astype(jnp.int32)
    wbits = lax.bitcast_convert_type(edge_weight.astype(jnp.float32),
                                     jnp.int32)

    pad = E_PAD - src.shape[0]
    zi = jnp.zeros((pad,), jnp.int32)
    src = jnp.concatenate([src, zi])
    dst = jnp.concatenate([dst, zi])
    wbits = jnp.concatenate([wbits, zi])

    packed = jnp.stack([src, dst, wbits], axis=0).reshape(3, -1, CH)
    packed = jnp.swapaxes(packed, 0, 1)
    packed = jnp.concatenate(
        [packed, jnp.zeros((2, 3, CH), jnp.int32)], axis=0)

    xw = _matmul(x, W)
    zeros = jnp.zeros((N, D), jnp.float32)
    partials = _sc_spmm(xw, packed, zeros)
    return _combine(partials, b.reshape(1, D))
